# baseline (speedup 1.0000x reference)
import jax
import jax.numpy as jnp
from jax import lax
from jax.experimental import pallas as pl
from jax.experimental.pallas import tpu as pltpu
from jax.experimental.pallas import tpu_sc as plsc

_NC, _NS, _L = 2, 16, 16
_NW = _NC * _NS
_NE = 8
_B = 256
_G = 64


def _wid():
    return lax.axis_index("s") * _NC + lax.axis_index("c")


def _sc_mesh():
    return plsc.VectorSubcoreMesh(core_axis_name="c", subcore_axis_name="s")


def _count_body(chunk, pol_hbm, counts_hbm, pol_v, cnt_v):
    wid = _wid()
    pltpu.sync_copy(pol_hbm.at[pl.ds(wid * chunk, chunk)], pol_v)

    def body(j, accs):
        p = pol_v[pl.ds(j * _L, _L)]
        return tuple(
            accs[e] + jnp.where(p == e, jnp.int32(1), jnp.int32(0))
            for e in range(_NE))

    accs = lax.fori_loop(0, chunk // _L, body,
                         tuple(jnp.zeros((_L,), jnp.int32) for _ in range(_NE)))
    for e in range(_NE):
        cnt_v[pl.ds(e * _L, _L)] = accs[e]
    pltpu.sync_copy(cnt_v, counts_hbm.at[wid])


def _routing_body(nblk_pad, cnt_ref, starts_ref, bexp_ref):
    nw, nc = cnt_ref.shape
    c = cnt_ref[...].astype(jnp.float32)
    dot = lambda a, b: jnp.dot(a, b, preferred_element_type=jnp.float32,
                               precision=lax.Precision.HIGHEST)

    wr = lax.broadcasted_iota(jnp.int32, (nw, nw), 0)
    wq = lax.broadcasted_iota(jnp.int32, (nw, nw), 1)

    kr = lax.broadcasted_iota(jnp.int32, (nc, nc), 0)
    kq = lax.broadcasted_iota(jnp.int32, (nc, nc), 1)
    same = (kr // _L) == (kq // _L)
    m_lane = (same & (kr < kq)).astype(jnp.float32)
    g_same = same.astype(jnp.float32)
    g_pre = ((kr // _L) < (kq // _L)).astype(jnp.float32)

    prefix_w = dot(dot((wq < wr).astype(jnp.float32), c), g_same)
    prefix_l = dot(c, m_lane)
    totrow = dot(jnp.ones((1, nw), jnp.float32), c)
    grp_tot = dot(totrow, g_same)
    nblk_row = jnp.floor((grp_tot + (_B - 1)) * (1.0 / _B))
    blkstart_row = dot(nblk_row, g_pre) * (1.0 / _L)
    starts = blkstart_row * float(_B) + prefix_w + prefix_l
    starts_ref[...] = starts.astype(jnp.int32)

    bi = lax.broadcasted_iota(jnp.int32, (1, nblk_pad), 1).astype(jnp.float32)
    bexp = jnp.zeros((1, nblk_pad), jnp.float32)
    for e in range(1, _NE):
        bs_e = blkstart_row[:, e * _L:e * _L + 1]
        bexp = bexp + (bi >= bs_e).astype(jnp.float32)
    bexp_ref[...] = bexp.astype(jnp.int32)


def _dispatch_body(chunk, dlat, dact, pol_hbm, lat_hbm, actp_hbm, starts_hbm,
                   xc_hbm, pos_hbm, pol_v, st_v, pos2_v, buf0_v, buf1_v,
                   lsem0, lsem1, ssem0, ssem1):
    wid = _wid()
    base = wid * chunk
    ng = chunk // _G
    bufs = (buf0_v, buf1_v)
    lsems = (lsem0, lsem1)
    ssems = (ssem0, ssem1)

    def load(g):
        b = bufs[g % 2]
        sm = lsems[g % 2]
        return (
            pltpu.async_copy(lat_hbm.at[pl.ds(base + g * _G, _G)],
                             b.at[:, pl.ds(0, dlat)], sm),
            pltpu.async_copy(actp_hbm.at[pl.ds(base + g * _G, _G)],
                             b.at[:, pl.ds(dlat, dact)], sm),
        )

    pltpu.sync_copy(pol_hbm.at[pl.ds(base, chunk)], pol_v)
    pltpu.sync_copy(starts_hbm.at[wid], st_v)
    ld = load(0)
    s = [st_v[pl.ds(e * _L, _L)] for e in range(_NE)]
    for j in range(chunk // _L):
        p = pol_v[pl.ds(j * _L, _L)]
        pos = jnp.zeros((_L,), jnp.int32)
        for e in range(_NE):
            m = p == e
            pos = jnp.where(m, s[e], pos)
            s[e] = s[e] + jnp.where(m, jnp.int32(1), jnp.int32(0))
        pos2_v.at[j // (_G // _L)][pl.ds((j % (_G // _L)) * _L, _L)] = pos
    pltpu.sync_copy(pos2_v, pos_hbm.at[pl.ds(wid * ng, ng)])
    scat = [None] * ng
    for g in range(ng):
        for c in ld:
            c.wait()
        if g + 1 < ng:
            if g >= 1:
                scat[g - 1].wait()
            ld = load(g + 1)
        scat[g] = pltpu.async_copy(bufs[g % 2], xc_hbm.at[pos2_v.at[g]],
                                   ssems[g % 2])
    scat[ng - 2].wait()
    scat[ng - 1].wait()


def _return_body(chunk, ys_hbm, pos_hbm, out_hbm, pos2_v, buf0_v, buf1_v,
                 gsem0, gsem1, wsem0, wsem1):
    wid = _wid()
    base = wid * chunk
    ng = chunk // _G
    bufs = (buf0_v, buf1_v)
    gsems = (gsem0, gsem1)
    wsems = (wsem0, wsem1)
    pltpu.sync_copy(pos_hbm.at[pl.ds(wid * ng, ng)], pos2_v)
    gath = [None] * ng
    wr = [None] * ng
    gath[0] = pltpu.async_copy(ys_hbm.at[pos2_v.at[0]], bufs[0], gsems[0])
    for g in range(ng):
        gath[g].wait()
        if g + 1 < ng:
            if g >= 1:
                wr[g - 1].wait()
            gath[g + 1] = pltpu.async_copy(ys_hbm.at[pos2_v.at[g + 1]],
                                           bufs[(g + 1) % 2], gsems[(g + 1) % 2])
        wr[g] = pltpu.async_copy(bufs[g % 2],
                                 out_hbm.at[pl.ds(base + g * _G, _G)],
                                 wsems[g % 2])
    wr[ng - 2].wait()
    wr[ng - 1].wait()


def _mm_body(bexp_ref, xc_ref, w1_ref, b1_ref, w2_ref, b2_ref, out_ref):
    x = xc_ref[...].astype(jnp.bfloat16)
    h = jnp.maximum(
        jnp.dot(x, w1_ref[0].astype(jnp.bfloat16),
                preferred_element_type=jnp.float32) + b1_ref[0], 0.0)
    out_ref[...] = (
        jnp.dot(h.astype(jnp.bfloat16), w2_ref[0].astype(jnp.bfloat16),
                preferred_element_type=jnp.float32) + b2_ref[0])


def kernel(latents, policy_indices, actions, W1, b1, W2, b2):
    N, D = latents.shape
    A = actions.shape[1]
    E, _, H = W1.shape
    chunk = N // _NW
    nblk = N // _B + E
    nblk_pad = ((nblk + _L - 1) // _L) * _L
    NP = nblk * _B

    DA = 128
    DP = D + DA
    pol = policy_indices.astype(jnp.int32)
    actp = jnp.pad(actions, ((0, 0), (0, DA - A)))
    W1c = jnp.concatenate(
        [W1[:, :D, :], jnp.pad(W1[:, D:, :], ((0, 0), (0, DA - A), (0, 0)))],
        axis=1)
    b1r = b1.reshape(E, 1, H)
    b2r = b2.reshape(E, 1, D)

    counts = pl.kernel(
        lambda *a: _count_body(chunk, *a),
        out_type=jax.ShapeDtypeStruct((_NW, _NE * _L), jnp.int32),
        mesh=_sc_mesh(),
        scratch_types=[
            pltpu.VMEM((chunk,), jnp.int32),
            pltpu.VMEM((_NE * _L,), jnp.int32),
        ],
    )(pol)

    starts, bexp = pl.pallas_call(
        lambda *a: _routing_body(nblk_pad, *a),
        out_shape=(jax.ShapeDtypeStruct((_NW, _NE * _L), jnp.int32),
                   jax.ShapeDtypeStruct((1, nblk_pad), jnp.int32)),
    )(counts)

    xc, pos = pl.kernel(
        lambda *a: _dispatch_body(chunk, D, DA, *a),
        out_type=(jax.ShapeDtypeStruct((NP, DP), jnp.float32),
                  jax.ShapeDtypeStruct((N // _G, _G), jnp.int32)),
        mesh=_sc_mesh(),
        scratch_types=[
            pltpu.VMEM((chunk,), jnp.int32),
            pltpu.VMEM((_NE * _L,), jnp.int32),
            pltpu.VMEM((chunk // _G, _G), jnp.int32),
            pltpu.VMEM((_G, DP), jnp.float32),
            pltpu.VMEM((_G, DP), jnp.float32),
            pltpu.SemaphoreType.DMA,
            pltpu.SemaphoreType.DMA,
            pltpu.SemaphoreType.DMA,
            pltpu.SemaphoreType.DMA,
        ],
    )(pol, latents, actp, starts)

    ys = pl.pallas_call(
        _mm_body,
        grid_spec=pltpu.PrefetchScalarGridSpec(
            num_scalar_prefetch=1,
            grid=(nblk,),
            in_specs=[
                pl.BlockSpec((_B, DP), lambda i, g: (i, 0)),
                pl.BlockSpec((1, DP, H), lambda i, g: (g[i], 0, 0)),
                pl.BlockSpec((1, 1, H), lambda i, g: (g[i], 0, 0)),
                pl.BlockSpec((1, H, D), lambda i, g: (g[i], 0, 0)),
                pl.BlockSpec((1, 1, D), lambda i, g: (g[i], 0, 0)),
            ],
            out_specs=pl.BlockSpec((_B, D), lambda i, g: (i, 0)),
        ),
        out_shape=jax.ShapeDtypeStruct((NP, D), jnp.float32),
    )(bexp.reshape(-1), xc, W1c, b1r, W2, b2r)

    next_latents = pl.kernel(
        lambda *a: _return_body(chunk, *a),
        out_type=jax.ShapeDtypeStruct((N, D), jnp.float32),
        mesh=_sc_mesh(),
        scratch_types=[
            pltpu.VMEM((chunk // _G, _G), jnp.int32),
            pltpu.VMEM((_G, D), jnp.float32),
            pltpu.VMEM((_G, D), jnp.float32),
            pltpu.SemaphoreType.DMA,
            pltpu.SemaphoreType.DMA,
            pltpu.SemaphoreType.DMA,
            pltpu.SemaphoreType.DMA,
        ],
    )(ys, pos)

    return next_latents


# speedup vs baseline: 1.1219x; 1.1219x over previous
import jax
import jax.numpy as jnp
from jax import lax
from jax.experimental import pallas as pl
from jax.experimental.pallas import tpu as pltpu
from jax.experimental.pallas import tpu_sc as plsc

_NC, _NS, _L = 2, 16, 16
_NW = _NC * _NS
_NE = 8
_B = 512
_G = 64


def _wid():
    return lax.axis_index("s") * _NC + lax.axis_index("c")


def _sc_mesh():
    return plsc.VectorSubcoreMesh(core_axis_name="c", subcore_axis_name="s")


def _count_body(chunk, pol_hbm, counts_hbm, pol_v, cnt_v):
    wid = _wid()
    pltpu.sync_copy(pol_hbm.at[pl.ds(wid * chunk, chunk)], pol_v)

    def body(j, accs):
        p = pol_v[pl.ds(j * _L, _L)]
        return tuple(
            accs[e] + jnp.where(p == e, jnp.int32(1), jnp.int32(0))
            for e in range(_NE))

    accs = lax.fori_loop(0, chunk // _L, body,
                         tuple(jnp.zeros((_L,), jnp.int32) for _ in range(_NE)))
    for e in range(_NE):
        cnt_v[pl.ds(e * _L, _L)] = accs[e]
    pltpu.sync_copy(cnt_v, counts_hbm.at[wid])


def _routing_body(nblk_pad, cnt_ref, starts_ref, bexp_ref):
    nw, nc = cnt_ref.shape
    c = cnt_ref[...].astype(jnp.float32)
    dot = lambda a, b: jnp.dot(a, b, preferred_element_type=jnp.float32,
                               precision=lax.Precision.HIGHEST)

    wr = lax.broadcasted_iota(jnp.int32, (nw, nw), 0)
    wq = lax.broadcasted_iota(jnp.int32, (nw, nw), 1)

    kr = lax.broadcasted_iota(jnp.int32, (nc, nc), 0)
    kq = lax.broadcasted_iota(jnp.int32, (nc, nc), 1)
    same = (kr // _L) == (kq // _L)
    m_lane = (same & (kr < kq)).astype(jnp.float32)
    g_same = same.astype(jnp.float32)
    g_pre = ((kr // _L) < (kq // _L)).astype(jnp.float32)

    prefix_w = dot(dot((wq < wr).astype(jnp.float32), c), g_same)
    prefix_l = dot(c, m_lane)
    totrow = dot(jnp.ones((1, nw), jnp.float32), c)
    grp_tot = dot(totrow, g_same)
    nblk_row = jnp.floor((grp_tot + (_B - 1)) * (1.0 / _B))
    blkstart_row = dot(nblk_row, g_pre) * (1.0 / _L)
    starts = blkstart_row * float(_B) + prefix_w + prefix_l
    starts_ref[...] = starts.astype(jnp.int32)

    bi = lax.broadcasted_iota(jnp.int32, (1, nblk_pad), 1).astype(jnp.float32)
    bexp = jnp.zeros((1, nblk_pad), jnp.float32)
    for e in range(1, _NE):
        bs_e = blkstart_row[:, e * _L:e * _L + 1]
        bexp = bexp + (bi >= bs_e).astype(jnp.float32)
    bexp_ref[...] = bexp.astype(jnp.int32)


def _dispatch_body(chunk, dlat, dact, pol_hbm, lat_hbm, actp_hbm, starts_hbm,
                   xc_hbm, pos_hbm, pol_v, st_v, pos2_v, buf0_v, buf1_v,
                   lsem0, lsem1, ssem0, ssem1):
    wid = _wid()
    base = wid * chunk
    ng = chunk // _G
    bufs = (buf0_v, buf1_v)
    lsems = (lsem0, lsem1)
    ssems = (ssem0, ssem1)

    def load(g):
        b = bufs[g % 2]
        sm = lsems[g % 2]
        return (
            pltpu.async_copy(lat_hbm.at[pl.ds(base + g * _G, _G)],
                             b.at[:, pl.ds(0, dlat)], sm),
            pltpu.async_copy(actp_hbm.at[pl.ds(base + g * _G, _G)],
                             b.at[:, pl.ds(dlat, dact)], sm),
        )

    pltpu.sync_copy(pol_hbm.at[pl.ds(base, chunk)], pol_v)
    pltpu.sync_copy(starts_hbm.at[wid], st_v)
    ld = load(0)
    s = [st_v[pl.ds(e * _L, _L)] for e in range(_NE)]
    for j in range(chunk // _L):
        p = pol_v[pl.ds(j * _L, _L)]
        pos = jnp.zeros((_L,), jnp.int32)
        for e in range(_NE):
            m = p == e
            pos = jnp.where(m, s[e], pos)
            s[e] = s[e] + jnp.where(m, jnp.int32(1), jnp.int32(0))
        pos2_v.at[j // (_G // _L)][pl.ds((j % (_G // _L)) * _L, _L)] = pos
    pltpu.sync_copy(pos2_v, pos_hbm.at[pl.ds(wid * ng, ng)])
    scat = [None] * ng
    for g in range(ng):
        for c in ld:
            c.wait()
        if g + 1 < ng:
            if g >= 1:
                scat[g - 1].wait()
            ld = load(g + 1)
        scat[g] = pltpu.async_copy(bufs[g % 2], xc_hbm.at[pos2_v.at[g]],
                                   ssems[g % 2])
    scat[ng - 2].wait()
    scat[ng - 1].wait()


def _return_body(chunk, ys_hbm, pos_hbm, out_hbm, pos2_v, buf0_v, buf1_v,
                 gsem0, gsem1, wsem0, wsem1):
    wid = _wid()
    base = wid * chunk
    ng = chunk // _G
    bufs = (buf0_v, buf1_v)
    gsems = (gsem0, gsem1)
    wsems = (wsem0, wsem1)
    pltpu.sync_copy(pos_hbm.at[pl.ds(wid * ng, ng)], pos2_v)
    gath = [None] * ng
    wr = [None] * ng
    gath[0] = pltpu.async_copy(ys_hbm.at[pos2_v.at[0]], bufs[0], gsems[0])
    for g in range(ng):
        gath[g].wait()
        if g + 1 < ng:
            if g >= 1:
                wr[g - 1].wait()
            gath[g + 1] = pltpu.async_copy(ys_hbm.at[pos2_v.at[g + 1]],
                                           bufs[(g + 1) % 2], gsems[(g + 1) % 2])
        wr[g] = pltpu.async_copy(bufs[g % 2],
                                 out_hbm.at[pl.ds(base + g * _G, _G)],
                                 wsems[g % 2])
    wr[ng - 2].wait()
    wr[ng - 1].wait()


def _mm_body(bexp_ref, xc_ref, w1_ref, b1_ref, w2_ref, b2_ref, out_ref):
    x = xc_ref[...].astype(jnp.bfloat16)
    h = jnp.maximum(
        jnp.dot(x, w1_ref[0].astype(jnp.bfloat16),
                preferred_element_type=jnp.float32) + b1_ref[0], 0.0)
    out_ref[...] = (
        jnp.dot(h.astype(jnp.bfloat16), w2_ref[0].astype(jnp.bfloat16),
                preferred_element_type=jnp.float32) + b2_ref[0])


def kernel(latents, policy_indices, actions, W1, b1, W2, b2):
    N, D = latents.shape
    A = actions.shape[1]
    E, _, H = W1.shape
    chunk = N // _NW
    nblk = N // _B + E
    nblk_pad = ((nblk + _L - 1) // _L) * _L
    NP = nblk * _B

    DA = 128
    DP = D + DA
    pol = policy_indices.astype(jnp.int32)
    actp = jnp.pad(actions, ((0, 0), (0, DA - A)))
    W1c = jnp.concatenate(
        [W1[:, :D, :], jnp.pad(W1[:, D:, :], ((0, 0), (0, DA - A), (0, 0)))],
        axis=1)
    b1r = b1.reshape(E, 1, H)
    b2r = b2.reshape(E, 1, D)

    counts = pl.kernel(
        lambda *a: _count_body(chunk, *a),
        out_type=jax.ShapeDtypeStruct((_NW, _NE * _L), jnp.int32),
        mesh=_sc_mesh(),
        scratch_types=[
            pltpu.VMEM((chunk,), jnp.int32),
            pltpu.VMEM((_NE * _L,), jnp.int32),
        ],
    )(pol)

    starts, bexp = pl.pallas_call(
        lambda *a: _routing_body(nblk_pad, *a),
        out_shape=(jax.ShapeDtypeStruct((_NW, _NE * _L), jnp.int32),
                   jax.ShapeDtypeStruct((1, nblk_pad), jnp.int32)),
    )(counts)

    xc, pos = pl.kernel(
        lambda *a: _dispatch_body(chunk, D, DA, *a),
        out_type=(jax.ShapeDtypeStruct((NP, DP), jnp.float32),
                  jax.ShapeDtypeStruct((N // _G, _G), jnp.int32)),
        mesh=_sc_mesh(),
        scratch_types=[
            pltpu.VMEM((chunk,), jnp.int32),
            pltpu.VMEM((_NE * _L,), jnp.int32),
            pltpu.VMEM((chunk // _G, _G), jnp.int32),
            pltpu.VMEM((_G, DP), jnp.float32),
            pltpu.VMEM((_G, DP), jnp.float32),
            pltpu.SemaphoreType.DMA,
            pltpu.SemaphoreType.DMA,
            pltpu.SemaphoreType.DMA,
            pltpu.SemaphoreType.DMA,
        ],
    )(pol, latents, actp, starts)

    ys = pl.pallas_call(
        _mm_body,
        grid_spec=pltpu.PrefetchScalarGridSpec(
            num_scalar_prefetch=1,
            grid=(nblk,),
            in_specs=[
                pl.BlockSpec((_B, DP), lambda i, g: (i, 0)),
                pl.BlockSpec((1, DP, H), lambda i, g: (g[i], 0, 0)),
                pl.BlockSpec((1, 1, H), lambda i, g: (g[i], 0, 0)),
                pl.BlockSpec((1, H, D), lambda i, g: (g[i], 0, 0)),
                pl.BlockSpec((1, 1, D), lambda i, g: (g[i], 0, 0)),
            ],
            out_specs=pl.BlockSpec((_B, D), lambda i, g: (i, 0)),
        ),
        out_shape=jax.ShapeDtypeStruct((NP, D), jnp.float32),
    )(bexp.reshape(-1), xc, W1c, b1r, W2, b2r)

    next_latents = pl.kernel(
        lambda *a: _return_body(chunk, *a),
        out_type=jax.ShapeDtypeStruct((N, D), jnp.float32),
        mesh=_sc_mesh(),
        scratch_types=[
            pltpu.VMEM((chunk // _G, _G), jnp.int32),
            pltpu.VMEM((_G, D), jnp.float32),
            pltpu.VMEM((_G, D), jnp.float32),
            pltpu.SemaphoreType.DMA,
            pltpu.SemaphoreType.DMA,
            pltpu.SemaphoreType.DMA,
            pltpu.SemaphoreType.DMA,
        ],
    )(ys, pos)

    return next_latents


# speedup vs baseline: 1.1691x; 1.0420x over previous
import jax
import jax.numpy as jnp
from jax import lax
from jax.experimental import pallas as pl
from jax.experimental.pallas import tpu as pltpu
from jax.experimental.pallas import tpu_sc as plsc

_NC, _NS, _L = 2, 16, 16
_NW = _NC * _NS
_NE = 8
_B = 1024
_G = 64


def _wid():
    return lax.axis_index("s") * _NC + lax.axis_index("c")


def _sc_mesh():
    return plsc.VectorSubcoreMesh(core_axis_name="c", subcore_axis_name="s")


def _count_body(chunk, pol_hbm, counts_hbm, pol_v, cnt_v):
    wid = _wid()
    pltpu.sync_copy(pol_hbm.at[pl.ds(wid * chunk, chunk)], pol_v)

    def body(j, accs):
        p = pol_v[pl.ds(j * _L, _L)]
        return tuple(
            accs[e] + jnp.where(p == e, jnp.int32(1), jnp.int32(0))
            for e in range(_NE))

    accs = lax.fori_loop(0, chunk // _L, body,
                         tuple(jnp.zeros((_L,), jnp.int32) for _ in range(_NE)))
    for e in range(_NE):
        cnt_v[pl.ds(e * _L, _L)] = accs[e]
    pltpu.sync_copy(cnt_v, counts_hbm.at[wid])


def _routing_body(nblk_pad, cnt_ref, starts_ref, bexp_ref):
    nw, nc = cnt_ref.shape
    c = cnt_ref[...].astype(jnp.float32)
    dot = lambda a, b: jnp.dot(a, b, preferred_element_type=jnp.float32,
                               precision=lax.Precision.HIGHEST)

    wr = lax.broadcasted_iota(jnp.int32, (nw, nw), 0)
    wq = lax.broadcasted_iota(jnp.int32, (nw, nw), 1)

    kr = lax.broadcasted_iota(jnp.int32, (nc, nc), 0)
    kq = lax.broadcasted_iota(jnp.int32, (nc, nc), 1)
    same = (kr // _L) == (kq // _L)
    m_lane = (same & (kr < kq)).astype(jnp.float32)
    g_same = same.astype(jnp.float32)
    g_pre = ((kr // _L) < (kq // _L)).astype(jnp.float32)

    prefix_w = dot(dot((wq < wr).astype(jnp.float32), c), g_same)
    prefix_l = dot(c, m_lane)
    totrow = dot(jnp.ones((1, nw), jnp.float32), c)
    grp_tot = dot(totrow, g_same)
    nblk_row = jnp.floor((grp_tot + (_B - 1)) * (1.0 / _B))
    blkstart_row = dot(nblk_row, g_pre) * (1.0 / _L)
    starts = blkstart_row * float(_B) + prefix_w + prefix_l
    starts_ref[...] = starts.astype(jnp.int32)

    bi = lax.broadcasted_iota(jnp.int32, (1, nblk_pad), 1).astype(jnp.float32)
    bexp = jnp.zeros((1, nblk_pad), jnp.float32)
    for e in range(1, _NE):
        bs_e = blkstart_row[:, e * _L:e * _L + 1]
        bexp = bexp + (bi >= bs_e).astype(jnp.float32)
    bexp_ref[...] = bexp.astype(jnp.int32)


def _dispatch_body(chunk, dlat, dact, pol_hbm, lat_hbm, actp_hbm, starts_hbm,
                   xc_hbm, pos_hbm, pol_v, st_v, pos2_v, buf0_v, buf1_v,
                   lsem0, lsem1, ssem0, ssem1):
    wid = _wid()
    base = wid * chunk
    ng = chunk // _G
    bufs = (buf0_v, buf1_v)
    lsems = (lsem0, lsem1)
    ssems = (ssem0, ssem1)

    def load(g):
        b = bufs[g % 2]
        sm = lsems[g % 2]
        return (
            pltpu.async_copy(lat_hbm.at[pl.ds(base + g * _G, _G)],
                             b.at[:, pl.ds(0, dlat)], sm),
            pltpu.async_copy(actp_hbm.at[pl.ds(base + g * _G, _G)],
                             b.at[:, pl.ds(dlat, dact)], sm),
        )

    pltpu.sync_copy(pol_hbm.at[pl.ds(base, chunk)], pol_v)
    pltpu.sync_copy(starts_hbm.at[wid], st_v)
    ld = load(0)
    s = [st_v[pl.ds(e * _L, _L)] for e in range(_NE)]
    for j in range(chunk // _L):
        p = pol_v[pl.ds(j * _L, _L)]
        pos = jnp.zeros((_L,), jnp.int32)
        for e in range(_NE):
            m = p == e
            pos = jnp.where(m, s[e], pos)
            s[e] = s[e] + jnp.where(m, jnp.int32(1), jnp.int32(0))
        pos2_v.at[j // (_G // _L)][pl.ds((j % (_G // _L)) * _L, _L)] = pos
    pltpu.sync_copy(pos2_v, pos_hbm.at[pl.ds(wid * ng, ng)])
    scat = [None] * ng
    for g in range(ng):
        for c in ld:
            c.wait()
        if g + 1 < ng:
            if g >= 1:
                scat[g - 1].wait()
            ld = load(g + 1)
        scat[g] = pltpu.async_copy(bufs[g % 2], xc_hbm.at[pos2_v.at[g]],
                                   ssems[g % 2])
    scat[ng - 2].wait()
    scat[ng - 1].wait()


def _return_body(chunk, ys_hbm, pos_hbm, out_hbm, pos2_v, buf0_v, buf1_v,
                 gsem0, gsem1, wsem0, wsem1):
    wid = _wid()
    base = wid * chunk
    ng = chunk // _G
    bufs = (buf0_v, buf1_v)
    gsems = (gsem0, gsem1)
    wsems = (wsem0, wsem1)
    pltpu.sync_copy(pos_hbm.at[pl.ds(wid * ng, ng)], pos2_v)
    gath = [None] * ng
    wr = [None] * ng
    gath[0] = pltpu.async_copy(ys_hbm.at[pos2_v.at[0]], bufs[0], gsems[0])
    for g in range(ng):
        gath[g].wait()
        if g + 1 < ng:
            if g >= 1:
                wr[g - 1].wait()
            gath[g + 1] = pltpu.async_copy(ys_hbm.at[pos2_v.at[g + 1]],
                                           bufs[(g + 1) % 2], gsems[(g + 1) % 2])
        wr[g] = pltpu.async_copy(bufs[g % 2],
                                 out_hbm.at[pl.ds(base + g * _G, _G)],
                                 wsems[g % 2])
    wr[ng - 2].wait()
    wr[ng - 1].wait()


def _mm_body(bexp_ref, xc_ref, w1_ref, b1_ref, w2_ref, b2_ref, out_ref):
    x = xc_ref[...].astype(jnp.bfloat16)
    h = jnp.maximum(
        jnp.dot(x, w1_ref[0].astype(jnp.bfloat16),
                preferred_element_type=jnp.float32) + b1_ref[0], 0.0)
    out_ref[...] = (
        jnp.dot(h.astype(jnp.bfloat16), w2_ref[0].astype(jnp.bfloat16),
                preferred_element_type=jnp.float32) + b2_ref[0])


def kernel(latents, policy_indices, actions, W1, b1, W2, b2):
    N, D = latents.shape
    A = actions.shape[1]
    E, _, H = W1.shape
    chunk = N // _NW
    nblk = N // _B + E
    nblk_pad = ((nblk + _L - 1) // _L) * _L
    NP = nblk * _B

    DA = 128
    DP = D + DA
    pol = policy_indices.astype(jnp.int32)
    actp = jnp.pad(actions, ((0, 0), (0, DA - A)))
    W1c = jnp.concatenate(
        [W1[:, :D, :], jnp.pad(W1[:, D:, :], ((0, 0), (0, DA - A), (0, 0)))],
        axis=1)
    b1r = b1.reshape(E, 1, H)
    b2r = b2.reshape(E, 1, D)

    counts = pl.kernel(
        lambda *a: _count_body(chunk, *a),
        out_type=jax.ShapeDtypeStruct((_NW, _NE * _L), jnp.int32),
        mesh=_sc_mesh(),
        scratch_types=[
            pltpu.VMEM((chunk,), jnp.int32),
            pltpu.VMEM((_NE * _L,), jnp.int32),
        ],
    )(pol)

    starts, bexp = pl.pallas_call(
        lambda *a: _routing_body(nblk_pad, *a),
        out_shape=(jax.ShapeDtypeStruct((_NW, _NE * _L), jnp.int32),
                   jax.ShapeDtypeStruct((1, nblk_pad), jnp.int32)),
    )(counts)

    xc, pos = pl.kernel(
        lambda *a: _dispatch_body(chunk, D, DA, *a),
        out_type=(jax.ShapeDtypeStruct((NP, DP), jnp.float32),
                  jax.ShapeDtypeStruct((N // _G, _G), jnp.int32)),
        mesh=_sc_mesh(),
        scratch_types=[
            pltpu.VMEM((chunk,), jnp.int32),
            pltpu.VMEM((_NE * _L,), jnp.int32),
            pltpu.VMEM((chunk // _G, _G), jnp.int32),
            pltpu.VMEM((_G, DP), jnp.float32),
            pltpu.VMEM((_G, DP), jnp.float32),
            pltpu.SemaphoreType.DMA,
            pltpu.SemaphoreType.DMA,
            pltpu.SemaphoreType.DMA,
            pltpu.SemaphoreType.DMA,
        ],
    )(pol, latents, actp, starts)

    ys = pl.pallas_call(
        _mm_body,
        grid_spec=pltpu.PrefetchScalarGridSpec(
            num_scalar_prefetch=1,
            grid=(nblk,),
            in_specs=[
                pl.BlockSpec((_B, DP), lambda i, g: (i, 0)),
                pl.BlockSpec((1, DP, H), lambda i, g: (g[i], 0, 0)),
                pl.BlockSpec((1, 1, H), lambda i, g: (g[i], 0, 0)),
                pl.BlockSpec((1, H, D), lambda i, g: (g[i], 0, 0)),
                pl.BlockSpec((1, 1, D), lambda i, g: (g[i], 0, 0)),
            ],
            out_specs=pl.BlockSpec((_B, D), lambda i, g: (i, 0)),
        ),
        out_shape=jax.ShapeDtypeStruct((NP, D), jnp.float32),
    )(bexp.reshape(-1), xc, W1c, b1r, W2, b2r)

    next_latents = pl.kernel(
        lambda *a: _return_body(chunk, *a),
        out_type=jax.ShapeDtypeStruct((N, D), jnp.float32),
        mesh=_sc_mesh(),
        scratch_types=[
            pltpu.VMEM((chunk // _G, _G), jnp.int32),
            pltpu.VMEM((_G, D), jnp.float32),
            pltpu.VMEM((_G, D), jnp.float32),
            pltpu.SemaphoreType.DMA,
            pltpu.SemaphoreType.DMA,
            pltpu.SemaphoreType.DMA,
            pltpu.SemaphoreType.DMA,
        ],
    )(ys, pos)

    return next_latents
